# Initial kernel scaffold; baseline (speedup 1.0000x reference)
import jax
import jax.numpy as jnp
from jax.experimental import pallas as pl


def kernel(batch_box_preds, batch_cls_preds):
    raise NotImplementedError("write your pallas kernel here")


# speedup vs baseline: 8.1999x; 8.1999x over previous
import functools

import jax
import jax.numpy as jnp
from jax.experimental import pallas as pl
from jax.experimental.pallas import tpu as pltpu

B, N, NUM_CLASS = 4, 20000, 3
PRE, POST, THRESH = 4096, 512, 0.7
BLK = 512
NBLK = PRE // BLK


def _nms_select_body(fields_ref, scores_ref, labels_ref, rois_ref, rsc_ref, rlb_ref,
                     valid_ref, kept_ref, cum_ref):
    f = fields_ref[0]
    sc = scores_ref[0]
    lb = labels_ref[0]

    cx = f[0:1]
    cy = f[1:2]
    dx = f[3:4]
    dy = f[4:5]
    x1 = cx - dx * 0.5
    x2 = cx + dx * 0.5
    y1 = cy - dy * 0.5
    y2 = cy + dy * 0.5
    areas = (x2 - x1) * (y2 - y1)

    valid_ref[...] = jnp.ones((1, PRE), jnp.float32)
    kept_ref[...] = jnp.zeros((1, PRE), jnp.float32)

    for b in range(NBLK):
        s = b * BLK
        nk = jnp.sum(kept_ref[...])

        @pl.when(nk < float(POST))
        def blk_body():
            fbT = jax.lax.transpose(f[:, s:s + BLK], (1, 0))
            x1c = fbT[:, 0:1] - fbT[:, 3:4] * 0.5
            x2c = fbT[:, 0:1] + fbT[:, 3:4] * 0.5
            y1c = fbT[:, 1:2] - fbT[:, 4:5] * 0.5
            y2c = fbT[:, 1:2] + fbT[:, 4:5] * 0.5
            areac = (x2c - x1c) * (y2c - y1c)

            x1b = x1[:, s:s + BLK]
            x2b = x2[:, s:s + BLK]
            y1b = y1[:, s:s + BLK]
            y2b = y2[:, s:s + BLK]
            areab = areas[:, s:s + BLK]
            vb = valid_ref[:, s:s + BLK]

            xx1 = jnp.maximum(x1c, x1b)
            yy1 = jnp.maximum(y1c, y1b)
            xx2 = jnp.minimum(x2c, x2b)
            yy2 = jnp.minimum(y2c, y2b)
            inter = jnp.clip(xx2 - xx1, 0.0) * jnp.clip(yy2 - yy1, 0.0)
            iou = inter / (areac + areab - inter + 1e-6)
            ii = jax.lax.broadcasted_iota(jnp.int32, (BLK, BLK), 0)
            jj = jax.lax.broadcasted_iota(jnp.int32, (BLK, BLK), 1)
            S = jnp.where((iou > THRESH) & (ii < jj), 1.0, 0.0)

            def fp_cond(c):
                return ~c[1]

            def fp_body(c):
                k, _ = c
                supp = jax.lax.dot_general(
                    k, S, (((1,), (0,)), ((), ())),
                    preferred_element_type=jnp.float32)
                k_new = vb * jnp.where(supp > 0.0, 0.0, 1.0)
                return (k_new, jnp.all(k_new == k))

            k0 = (vb, jnp.array(False))
            kb, _ = jax.lax.while_loop(fp_cond, fp_body, k0)

            kept_ref[:, s:s + BLK] = kb

            if b < NBLK - 1:
                kc = jax.lax.transpose(kb, (1, 0))
                for jc in range(b + 1, NBLK):
                    t = jc * BLK
                    xx1 = jnp.maximum(x1c, x1[:, t:t + BLK])
                    yy1 = jnp.maximum(y1c, y1[:, t:t + BLK])
                    xx2 = jnp.minimum(x2c, x2[:, t:t + BLK])
                    yy2 = jnp.minimum(y2c, y2[:, t:t + BLK])
                    inter = jnp.clip(xx2 - xx1, 0.0) * jnp.clip(yy2 - yy1, 0.0)
                    iou = inter / (areac + areas[:, t:t + BLK] - inter + 1e-6)
                    hit = jnp.where((iou > THRESH) & (kc > 0.0), 1.0, 0.0)
                    supp = jnp.max(hit, axis=0, keepdims=True)
                    valid_ref[:, t:t + BLK] = valid_ref[:, t:t + BLK] * (1.0 - supp)

    kept = kept_ref[...]
    li = jax.lax.broadcasted_iota(jnp.int32, (BLK, BLK), 0)
    lj = jax.lax.broadcasted_iota(jnp.int32, (BLK, BLK), 1)
    Lstrict = jnp.where(li < lj, 1.0, 0.0)
    carry = jnp.zeros((1, 1), jnp.float32)
    for c in range(NBLK):
        s = c * BLK
        kc = kept[:, s:s + BLK]
        cc = jax.lax.dot_general(
            kc, Lstrict, (((1,), (0,)), ((), ())),
            preferred_element_type=jnp.float32) + carry
        cum_ref[:, s:s + BLK] = cc
        carry = carry + jnp.sum(kc).reshape(1, 1)

    cum = cum_ref[...].astype(jnp.int32)
    slot = jax.lax.broadcasted_iota(jnp.int32, (POST, PRE), 0)
    OH = jnp.where((slot == cum) & (kept > 0.0), 1.0, 0.0)

    rois = jax.lax.dot_general(
        OH, f, (((1,), (1,)), ((), ())), preferred_element_type=jnp.float32)
    rsc = jax.lax.dot_general(
        OH, sc, (((1,), (1,)), ((), ())), preferred_element_type=jnp.float32)
    lbf = lb.astype(jnp.float32) + 1.0
    rlb = jax.lax.dot_general(
        OH, lbf, (((1,), (1,)), ((), ())), preferred_element_type=jnp.float32)

    rois_ref[0] = rois
    rsc_ref[0] = rsc
    rlb_ref[0] = rlb.astype(jnp.int32)


def _nms_select(fields, scores, labels):
    return pl.pallas_call(
        _nms_select_body,
        grid=(B,),
        in_specs=[
            pl.BlockSpec((1, 8, PRE), lambda b: (b, 0, 0)),
            pl.BlockSpec((1, 1, PRE), lambda b: (b, 0, 0)),
            pl.BlockSpec((1, 1, PRE), lambda b: (b, 0, 0)),
        ],
        out_specs=[
            pl.BlockSpec((1, POST, 8), lambda b: (b, 0, 0)),
            pl.BlockSpec((1, POST, 1), lambda b: (b, 0, 0)),
            pl.BlockSpec((1, POST, 1), lambda b: (b, 0, 0)),
        ],
        out_shape=[
            jax.ShapeDtypeStruct((B, POST, 8), jnp.float32),
            jax.ShapeDtypeStruct((B, POST, 1), jnp.float32),
            jax.ShapeDtypeStruct((B, POST, 1), jnp.int32),
        ],
        scratch_shapes=[
            pltpu.VMEM((1, PRE), jnp.float32),
            pltpu.VMEM((1, PRE), jnp.float32),
            pltpu.VMEM((1, PRE), jnp.float32),
        ],
    )(fields, scores, labels)


def kernel(batch_box_preds, batch_cls_preds):
    scores_all = jnp.max(batch_cls_preds, axis=-1)
    labels_all = jnp.argmax(batch_cls_preds, axis=-1).astype(jnp.int32)

    top_scores, order = jax.lax.top_k(scores_all, PRE)
    boxes_sorted = jnp.take_along_axis(batch_box_preds, order[..., None], axis=1)
    labels_sorted = jnp.take_along_axis(labels_all, order, axis=1)

    fields = jnp.concatenate(
        [jnp.transpose(boxes_sorted, (0, 2, 1)),
         jnp.zeros((B, 1, PRE), jnp.float32)], axis=1)
    scores_in = top_scores[:, None, :]
    labels_in = labels_sorted[:, None, :]

    rois8, rsc, rlb = _nms_select(fields, scores_in, labels_in)
    return rois8[:, :, :7], rsc[..., 0], rlb[..., 0]
